# baseline (speedup 1.0000x reference)
import functools

import jax
import jax.numpy as jnp
from jax import lax
from jax.experimental import pallas as pl
from jax.experimental.pallas import tpu as pltpu
from jax.experimental.pallas import tpu_sc as plsc

EMBED = 128
NUM_CTX = 5
LANES = 16
NSLICE = EMBED // LANES
CHUNK = 64


@functools.lru_cache(maxsize=None)
def _make_kernel(B):
    info = plsc.get_sparse_core_info()
    NC, NS = info.num_cores, info.num_subcores
    NW = NC * NS
    items = B // NW
    nchunk = items // CHUNK

    mesh = plsc.VectorSubcoreMesh(core_axis_name="c", subcore_axis_name="s")

    scratch = []
    scratch += [pltpu.VMEM((CHUNK,), jnp.int32) for _ in range(2)]
    scratch += [pltpu.VMEM((CHUNK,), jnp.int32)
                for _ in range(NUM_CTX * 2)]
    scratch += [pltpu.VMEM((CHUNK, EMBED), jnp.float32)
                for _ in range(2)]
    scratch += [pltpu.VMEM((CHUNK, EMBED), jnp.float32)
                for _ in range(NUM_CTX * 2)]
    scratch.append(pltpu.VMEM((NUM_CTX, items), jnp.float32))
    scratch.append(pltpu.SemaphoreType.DMA)
    scratch.append(pltpu.SemaphoreType.DMA)

    @functools.partial(
        pl.kernel,
        mesh=mesh,
        out_type=jax.ShapeDtypeStruct((NUM_CTX, B), jnp.float32),
        scratch_types=scratch,
        compiler_params=pltpu.CompilerParams(needs_layout_passes=False),
    )
    def k(tgt_hbm, ctx_hbm, wt_hbm, wc_hbm, out_hbm, *sc):
        it = iter(sc)
        t_gidx = [next(it) for _ in range(2)]
        c_gidx = [[next(it) for _ in range(2)] for _ in range(NUM_CTX)]
        t_rows = [next(it) for _ in range(2)]
        c_rows = [[next(it) for _ in range(2)] for _ in range(NUM_CTX)]
        out_v = next(it)
        sems = [next(it), next(it)]

        wid = lax.axis_index("s") * NC + lax.axis_index("c")
        wbase = wid * items

        def issue(g, p):
            base = wbase + g * CHUNK
            pltpu.sync_copy(tgt_hbm.at[pl.ds(base, CHUNK)], t_gidx[p])
            for c in range(NUM_CTX):
                pltpu.sync_copy(ctx_hbm.at[c, pl.ds(base, CHUNK)],
                                c_gidx[c][p])
            handles = [pltpu.async_copy(
                wt_hbm.at[t_gidx[p]], t_rows[p], sems[p])]
            for c in range(NUM_CTX):
                handles.append(pltpu.async_copy(
                    wc_hbm.at[c_gidx[c][p]], c_rows[c][p], sems[p]))
            return handles

        lanes = lax.iota(jnp.int32, LANES)

        out_mask = lanes < NUM_CTX

        def compute(g, p):
            last = jnp.full((LANES,), LANES - 1, jnp.int32)

            @plsc.parallel_loop(0, CHUNK, unroll=4)
            def body(i):
                tr = t_rows[p]
                accs = [None] * NUM_CTX
                for s in range(NSLICE):
                    sl = pl.ds(s * LANES, LANES)
                    t_s = tr[i, sl]
                    for c in range(NUM_CTX):
                        prod = t_s * c_rows[c][p][i, sl]
                        accs[c] = prod if s == 0 else accs[c] + prod
                res = jnp.zeros((LANES,), jnp.float32)
                for c in range(NUM_CTX):
                    d = jnp.sum(accs[c])
                    res = jnp.where(lanes == c, jnp.full((LANES,), d), res)
                ocol = jnp.full((LANES,), g * CHUNK + i, jnp.int32)
                plsc.store_scatter(out_v, [lanes, ocol], res, mask=out_mask)

        handles = issue(0, 0)
        for g in range(nchunk):
            p = g % 2
            nxt = issue(g + 1, 1 - p) if g + 1 < nchunk else None
            for h in handles:
                h.wait()
            compute(g, p)
            handles = nxt

        for c in range(NUM_CTX):
            pltpu.sync_copy(out_v.at[pl.ds(c, 1), pl.ds(0, items)],
                            out_hbm.at[pl.ds(c, 1), pl.ds(wbase, items)])

    return k


@jax.jit
def kernel(target, context, W_target, W_context):
    B = target.shape[0]
    tgt = target.astype(jnp.int32)
    ctx_t = jnp.transpose(context.astype(jnp.int32))
    dots_t = _make_kernel(B)(tgt, ctx_t, W_target, W_context)
    return jnp.transpose(dots_t)


# speedup vs baseline: 1.4959x; 1.4959x over previous
import functools

import jax
import jax.numpy as jnp
from jax import lax
from jax.experimental import pallas as pl
from jax.experimental.pallas import tpu as pltpu
from jax.experimental.pallas import tpu_sc as plsc

EMBED = 128
NUM_CTX = 5
LANES = 16
NSLICE = EMBED // LANES
CHUNK = 64


@functools.lru_cache(maxsize=None)
def _make_kernel(B):
    info = plsc.get_sparse_core_info()
    NC, NS = info.num_cores, info.num_subcores
    NW = NC * NS
    items = B // NW
    nchunk = items // CHUNK

    mesh = plsc.VectorSubcoreMesh(core_axis_name="c", subcore_axis_name="s")

    NIDX = 3

    scratch = []
    scratch += [pltpu.VMEM((CHUNK,), jnp.int32) for _ in range(NIDX)]
    scratch += [pltpu.VMEM((CHUNK,), jnp.int32)
                for _ in range(NUM_CTX * NIDX)]
    scratch += [pltpu.VMEM((CHUNK, EMBED), jnp.float32)
                for _ in range(2)]
    scratch += [pltpu.VMEM((CHUNK, EMBED), jnp.float32)
                for _ in range(NUM_CTX * 2)]
    scratch.append(pltpu.VMEM((NUM_CTX, items), jnp.float32))
    scratch.append(pltpu.SemaphoreType.DMA)
    scratch.append(pltpu.SemaphoreType.DMA)
    scratch.append(pltpu.SemaphoreType.DMA)

    @functools.partial(
        pl.kernel,
        mesh=mesh,
        out_type=jax.ShapeDtypeStruct((NUM_CTX, B), jnp.float32),
        scratch_types=scratch,
        compiler_params=pltpu.CompilerParams(needs_layout_passes=False),
    )
    def k(tgt_hbm, ctx_hbm, wt_hbm, wc_hbm, out_hbm, *sc):
        it = iter(sc)
        t_gidx = [next(it) for _ in range(NIDX)]
        c_gidx = [[next(it) for _ in range(NIDX)] for _ in range(NUM_CTX)]
        t_rows = [next(it) for _ in range(2)]
        c_rows = [[next(it) for _ in range(2)] for _ in range(NUM_CTX)]
        out_v = next(it)
        sems = [next(it), next(it)]
        isem = next(it)

        wid = lax.axis_index("s") * NC + lax.axis_index("c")
        wbase = wid * items

        def fetch_idx(g):
            q = g % NIDX
            base = wbase + g * CHUNK
            handles = [pltpu.async_copy(
                tgt_hbm.at[pl.ds(base, CHUNK)], t_gidx[q], isem)]
            for c in range(NUM_CTX):
                handles.append(pltpu.async_copy(
                    ctx_hbm.at[c, pl.ds(base, CHUNK)], c_gidx[c][q], isem))
            return handles

        def issue(g, p):
            q = g % NIDX
            handles = [pltpu.async_copy(
                wt_hbm.at[t_gidx[q]], t_rows[p], sems[p])]
            for c in range(NUM_CTX):
                handles.append(pltpu.async_copy(
                    wc_hbm.at[c_gidx[c][q]], c_rows[c][p], sems[p]))
            return handles

        lanes = lax.iota(jnp.int32, LANES)

        out_mask = lanes < NUM_CTX

        def compute(g, p):
            @plsc.parallel_loop(0, CHUNK, unroll=2)
            def body(i):
                tr = t_rows[p]
                t = [tr[i, pl.ds(s * LANES, LANES)] for s in range(NSLICE)]
                res = jnp.zeros((LANES,), jnp.float32)
                for c in range(NUM_CTX):
                    cr = c_rows[c][p]
                    acc = t[0] * cr[i, pl.ds(0, LANES)]
                    for s in range(1, NSLICE):
                        acc = acc + t[s] * cr[i, pl.ds(s * LANES, LANES)]
                    d = jnp.sum(acc)
                    res = jnp.where(lanes == c, jnp.full((LANES,), d), res)
                ocol = jnp.full((LANES,), g * CHUNK + i, jnp.int32)
                plsc.store_scatter(out_v, [lanes, ocol], res, mask=out_mask)

        ih = fetch_idx(0)
        for h in ih:
            h.wait()
        handles = issue(0, 0)
        ih_next = fetch_idx(1) if nchunk > 1 else None
        for g in range(nchunk):
            p = g % 2
            nxt = None
            if g + 1 < nchunk:
                for h in ih_next:
                    h.wait()
                nxt = issue(g + 1, 1 - p)
                ih_next = fetch_idx(g + 2) if g + 2 < nchunk else None
            for h in handles:
                h.wait()
            compute(g, p)
            handles = nxt

        for c in range(NUM_CTX):
            pltpu.sync_copy(out_v.at[pl.ds(c, 1), pl.ds(0, items)],
                            out_hbm.at[pl.ds(c, 1), pl.ds(wbase, items)])

    return k


@jax.jit
def kernel(target, context, W_target, W_context):
    B = target.shape[0]
    tgt = target.astype(jnp.int32)
    ctx_t = jnp.transpose(context.astype(jnp.int32))
    dots_t = _make_kernel(B)(tgt, ctx_t, W_target, W_context)
    return jnp.transpose(dots_t)
